# baseline (speedup 1.0000x reference)
import functools

import jax
import jax.numpy as jnp
import numpy as np
from jax.experimental import pallas as pl
from jax.experimental.pallas import tpu as pltpu

B = 32
N = 1000000
BLK = 8192
NB = (N + BLK - 1) // BLK

_TINY = np.float32(np.finfo(np.float32).tiny)
_K0 = np.uint32(0)
_K1 = np.uint32(42)
_K2 = np.uint32(np.uint32(0) ^ np.uint32(42) ^ np.uint32(0x1BD11BDA))
_KS = (_K0, _K1, _K2)
_ROT = ((13, 15, 26, 6), (17, 29, 16, 24))
_INJ = ((1, 2), (2, 0), (0, 1), (1, 2), (2, 0))


def _rotl(x, r):
    return (x << np.uint32(r)) | (x >> np.uint32(32 - r))


def _threefry_bits(x0, x1):
    x0 = x0 + _KS[0]
    x1 = x1 + _KS[1]
    for g in range(5):
        for r in _ROT[g % 2]:
            x0 = x0 + x1
            x1 = _rotl(x1, r)
            x1 = x1 ^ x0
        a, b = _INJ[g]
        x0 = x0 + _KS[a]
        x1 = x1 + (_KS[b] + np.uint32(g + 1))
    return x0 ^ x1


def _gumbel_from_bits(bits):
    fb = (bits >> np.uint32(9)) | np.uint32(0x3F800000)
    fl = jax.lax.bitcast_convert_type(fb, jnp.float32) - jnp.float32(1.0)
    u = jnp.maximum(fl, _TINY)
    return -jnp.log(-jnp.log(u))


def _body(x_ref, act_ref, nlp_ref,
          pmax_ref, idx_ref, pick_ref, m_ref, s_ref):
    c = pl.program_id(0)

    @pl.when(c == 0)
    def _init():
        pmax_ref[...] = jnp.full((B,), -jnp.inf, jnp.float32)
        idx_ref[...] = jnp.zeros((B,), jnp.int32)
        pick_ref[...] = jnp.zeros((B,), jnp.float32)
        m_ref[...] = jnp.full((B,), -jnp.inf, jnp.float32)
        s_ref[...] = jnp.zeros((B,), jnp.float32)

    x = x_ref[...]
    col = c * BLK + jax.lax.broadcasted_iota(jnp.int32, (B, BLK), 1)
    valid = col < N
    row = jax.lax.broadcasted_iota(jnp.uint32, (B, BLK), 0)
    flat = row * np.uint32(N) + col.astype(jnp.uint32)
    bits = _threefry_bits(jnp.zeros((B, BLK), jnp.uint32), flat)
    g = _gumbel_from_bits(bits)

    p = jnp.where(valid, x + g, -jnp.inf)
    bpm = jnp.max(p, axis=1)
    cand = jnp.where(p == bpm[:, None], col, jnp.int32(2**30))
    bidx = jnp.min(cand, axis=1)
    bpick = jnp.max(jnp.where(col == bidx[:, None], x, -jnp.inf), axis=1)

    xm = jnp.where(valid, x, -jnp.inf)
    bm = jnp.max(xm, axis=1)
    bs = jnp.sum(jnp.where(valid, jnp.exp(x - bm[:, None]), 0.0), axis=1)

    old = pmax_ref[...]
    better = bpm > old
    pmax_ref[...] = jnp.where(better, bpm, old)
    idx_ref[...] = jnp.where(better, bidx, idx_ref[...])
    pick_ref[...] = jnp.where(better, bpick, pick_ref[...])

    mo = m_ref[...]
    so = s_ref[...]
    mn = jnp.maximum(mo, bm)
    s_ref[...] = so * jnp.exp(mo - mn) + bs * jnp.exp(bm - mn)
    m_ref[...] = mn

    @pl.when(c == NB - 1)
    def _fin():
        act_ref[...] = idx_ref[...]
        nlp_ref[...] = (m_ref[...] + jnp.log(s_ref[...])) - pick_ref[...]


@jax.jit
def kernel(logits):
    action, neglogprob = pl.pallas_call(
        _body,
        grid=(NB,),
        in_specs=[pl.BlockSpec((B, BLK), lambda c: (0, c))],
        out_specs=[
            pl.BlockSpec((B,), lambda c: (0,)),
            pl.BlockSpec((B,), lambda c: (0,)),
        ],
        out_shape=[
            jax.ShapeDtypeStruct((B,), jnp.int32),
            jax.ShapeDtypeStruct((B,), jnp.float32),
        ],
        scratch_shapes=[
            pltpu.VMEM((B,), jnp.float32),
            pltpu.VMEM((B,), jnp.int32),
            pltpu.VMEM((B,), jnp.float32),
            pltpu.VMEM((B,), jnp.float32),
            pltpu.VMEM((B,), jnp.float32),
        ],
    )(logits)
    return action, neglogprob
